# baseline (speedup 1.0000x reference)
import functools

import jax
import jax.numpy as jnp
from jax import lax
from jax.experimental import pallas as pl
from jax.experimental.pallas import tpu as pltpu
from jax.experimental.pallas import tpu_sc as plsc

N = 1_000_000
LANES = 16
NW = 32
CHUNK_G = 250
CHUNK_E = CHUNK_G * LANES
CHUNK_O = CHUNK_E * 5
NCHUNKS = N // CHUNK_E
MAX_ROUNDS = -(-NCHUNKS // NW)


def _onehot_kernel(elems_hbm, out_hbm, e_buf, o_buf):
    c = lax.axis_index("c")
    s = lax.axis_index("s")
    wid = s * 2 + c

    lane = lax.broadcasted_iota(jnp.int32, (LANES,), 0)
    div_c = []
    typ_c = []
    for j in range(5):
        f = lane + (16 * j)
        d = f // 5
        a = f - d * 5
        div_c.append(d)
        typ_c.append(jnp.where(a == 0, 1, a + 5).astype(jnp.int32))

    one = jnp.full((LANES,), 1.0, dtype=jnp.float32)
    zero = jnp.zeros((LANES,), dtype=jnp.float32)

    for r in range(MAX_ROUNDS):
        chunk = wid + r * NW

        @pl.when(chunk < NCHUNKS)
        def _process():
            base_e = pl.multiple_of(chunk * CHUNK_E, 8)
            base_o = pl.multiple_of(chunk * CHUNK_O, 8)
            pltpu.sync_copy(elems_hbm.at[pl.ds(base_e, CHUNK_E)], e_buf)

            def grp(g, carry):
                g16 = g * LANES
                g80 = g * (LANES * 5)
                for j in range(5):
                    e = plsc.load_gather(e_buf, [div_c[j] + g16])
                    o_buf[pl.ds(g80 + j * LANES, LANES)] = jnp.where(
                        e == typ_c[j], one, zero)
                return carry

            lax.fori_loop(0, CHUNK_G, grp, 0, unroll=2)
            pltpu.sync_copy(o_buf, out_hbm.at[pl.ds(base_o, CHUNK_O)])


def kernel(elems):
    mesh = plsc.VectorSubcoreMesh(core_axis_name="c", subcore_axis_name="s")
    run = functools.partial(
        pl.kernel,
        mesh=mesh,
        out_type=jax.ShapeDtypeStruct((N * 5,), jnp.float32),
        scratch_types=[
            pltpu.VMEM((CHUNK_E,), jnp.int32),
            pltpu.VMEM((CHUNK_O,), jnp.float32),
        ],
        compiler_params=pltpu.CompilerParams(needs_layout_passes=False),
    )(_onehot_kernel)
    out = run(elems.astype(jnp.int32))
    return out.reshape(N, 5)


# speedup vs baseline: 10.7925x; 10.7925x over previous
import jax
import jax.numpy as jnp
from jax import lax
from jax.experimental import pallas as pl

N = 1_000_000
B = 8192


def _onehot_body(e_ref, o_ref):
    e = e_ref[...]
    row = lax.broadcasted_iota(jnp.int32, (5, B), 0)
    t = jnp.where(row == 0, 1, row + 5)
    eb = jnp.broadcast_to(e[None, :], (5, B))
    o_ref[...] = jnp.where(eb == t,
                           jnp.ones((5, B), jnp.float32),
                           jnp.zeros((5, B), jnp.float32))


def kernel(elems):
    grid = (N + B - 1) // B
    out = pl.pallas_call(
        _onehot_body,
        grid=(grid,),
        in_specs=[pl.BlockSpec((B,), lambda i: (i,))],
        out_specs=pl.BlockSpec((5, B), lambda i: (0, i)),
        out_shape=jax.ShapeDtypeStruct((5, N), jnp.float32),
    )(elems.astype(jnp.int32))
    return out.T
